# Initial kernel scaffold; baseline (speedup 1.0000x reference)
import jax
import jax.numpy as jnp
from jax.experimental import pallas as pl


def kernel(h, teacher_id, materia_id, teacher_emb, materia_emb, W, b):
    raise NotImplementedError("write your pallas kernel here")


# speedup vs baseline: 1.5776x; 1.5776x over previous
import functools

import jax
import jax.numpy as jnp
from jax import lax
from jax.experimental import pallas as pl
from jax.experimental.pallas import tpu as pltpu
from jax.experimental.pallas import tpu_sc as plsc

N_HIDDEN = 128
EMB = 16
BATCH = 16384
NW = 32
BPW = BATCH // NW
CHUNK = 128
NCH = BPW // CHUNK
BS = 2048


def _sc_gather(t_emb, m_emb, tid2, mid2):
    mesh = plsc.VectorSubcoreMesh(core_axis_name="c", subcore_axis_name="s")

    @functools.partial(
        pl.kernel,
        mesh=mesh,
        compiler_params=pltpu.CompilerParams(use_tc_tiling_on_sc=False),
        out_type=(
            jax.ShapeDtypeStruct((BATCH, EMB), jnp.float32),
            jax.ShapeDtypeStruct((BATCH, EMB), jnp.float32),
        ),
        scratch_types=[
            pltpu.VMEM((NCH, CHUNK), jnp.int32),
            pltpu.VMEM((NCH, CHUNK), jnp.int32),
            pltpu.VMEM((BPW, EMB), jnp.float32),
            pltpu.VMEM((BPW, EMB), jnp.float32),
            pltpu.SemaphoreType.DMA,
            pltpu.SemaphoreType.DMA,
        ],
    )
    def k(t_emb_hbm, m_emb_hbm, tid_hbm, mid_hbm, t_out, m_out,
          tid_v, mid_v, t_rows, m_rows, sem_t, sem_m):
        wid = lax.axis_index("s") * 2 + lax.axis_index("c")
        base = wid * BPW
        pltpu.sync_copy(tid_hbm.at[pl.ds(wid * NCH, NCH)], tid_v)
        pltpu.sync_copy(mid_hbm.at[pl.ds(wid * NCH, NCH)], mid_v)
        copies = []
        for j in range(NCH):
            copies.append(pltpu.async_copy(
                t_emb_hbm.at[tid_v.at[j]],
                t_rows.at[pl.ds(j * CHUNK, CHUNK)], sem_t))
            copies.append(pltpu.async_copy(
                m_emb_hbm.at[mid_v.at[j]],
                m_rows.at[pl.ds(j * CHUNK, CHUNK)], sem_m))
        for c in copies:
            c.wait()
        pltpu.sync_copy(t_rows, t_out.at[pl.ds(base, BPW)])
        pltpu.sync_copy(m_rows, m_out.at[pl.ds(base, BPW)])

    return k(t_emb, m_emb, tid2, mid2)


def _tc_body(b_ref, wh_ref, wt_ref, wm_ref, h_ref, t_ref, m_ref, o_ref):
    yh = jnp.sum(h_ref[...] * wh_ref[...], axis=1)
    yt = jnp.sum(t_ref[...] * wt_ref[...], axis=1)
    ym = jnp.sum(m_ref[...] * wm_ref[...], axis=1)
    o_ref[...] = yh + yt + ym + b_ref[0]


def _tc_head(h, t_rows, m_rows, wh, wt, wm, b):
    return pl.pallas_call(
        _tc_body,
        grid=(BATCH // BS,),
        in_specs=[
            pl.BlockSpec(memory_space=pltpu.SMEM),
            pl.BlockSpec((1, N_HIDDEN), lambda i: (0, 0)),
            pl.BlockSpec((1, EMB), lambda i: (0, 0)),
            pl.BlockSpec((1, EMB), lambda i: (0, 0)),
            pl.BlockSpec((BS, N_HIDDEN), lambda i: (i, 0)),
            pl.BlockSpec((BS, EMB), lambda i: (i, 0)),
            pl.BlockSpec((BS, EMB), lambda i: (i, 0)),
        ],
        out_specs=pl.BlockSpec((BS,), lambda i: (i,)),
        out_shape=jax.ShapeDtypeStruct((BATCH,), jnp.float32),
    )(b, wh, wt, wm, h, t_rows, m_rows)


def kernel(h, teacher_id, materia_id, teacher_emb, materia_emb, W, b):
    tid2 = teacher_id.astype(jnp.int32).reshape(NW * NCH, CHUNK)
    mid2 = materia_id.astype(jnp.int32).reshape(NW * NCH, CHUNK)
    t_rows, m_rows = _sc_gather(teacher_emb, materia_emb, tid2, mid2)
    wh = W[:, :N_HIDDEN]
    wt = W[:, N_HIDDEN:N_HIDDEN + EMB]
    wm = W[:, N_HIDDEN + EMB:]
    return _tc_head(h, t_rows, m_rows, wh, wt, wm, b)


# speedup vs baseline: 3.2499x; 2.0600x over previous
import functools

import jax
import jax.numpy as jnp
from jax import lax
from jax.experimental import pallas as pl
from jax.experimental.pallas import tpu as pltpu
from jax.experimental.pallas import tpu_sc as plsc

N_HIDDEN = 128
EMB = 16
BATCH = 16384
N_TEACH = 100000
N_MAT = 1000
N_MAT_PAD = 1024
NW = 32
BPW = BATCH // NW
CHUNK = 128
NCH = BPW // CHUNK
BSC = 8192
BS = 2048


def _score_body(wc_ref, tT_ref, o_ref):
    o_ref[...] = jnp.sum(tT_ref[...] * wc_ref[...], axis=0)


def _scores(tT, wc, n, blk):
    grid = (n + blk - 1) // blk
    return pl.pallas_call(
        _score_body,
        grid=(grid,),
        in_specs=[
            pl.BlockSpec((EMB, 1), lambda i: (0, 0)),
            pl.BlockSpec((EMB, blk), lambda i: (0, i)),
        ],
        out_specs=pl.BlockSpec((blk,), lambda i: (i,)),
        out_shape=jax.ShapeDtypeStruct((n,), jnp.float32),
    )(wc, tT)


def _sc_gather_scores(score_t, score_m, tid, mid):
    mesh = plsc.VectorSubcoreMesh(core_axis_name="c", subcore_axis_name="s")

    @functools.partial(
        pl.kernel,
        mesh=mesh,
        compiler_params=pltpu.CompilerParams(use_tc_tiling_on_sc=False),
        out_type=jax.ShapeDtypeStruct((BATCH,), jnp.float32),
        scratch_types=[
            pltpu.VMEM((BPW,), jnp.int32),
            pltpu.VMEM((BPW,), jnp.int32),
            pltpu.VMEM((BPW,), jnp.float32),
            pltpu.VMEM((BPW,), jnp.float32),
            pltpu.VMEM((BPW,), jnp.float32),
            pltpu.SemaphoreType.DMA,
        ],
    )
    def k(st_hbm, sm_hbm, tid_hbm, mid_hbm, out_hbm,
          tid_v, mid_v, tval_v, mval_v, y_v, sem):
        wid = lax.axis_index("s") * 2 + lax.axis_index("c")
        base = wid * BPW
        pltpu.sync_copy(tid_hbm.at[pl.ds(base, BPW)], tid_v)
        pltpu.sync_copy(mid_hbm.at[pl.ds(base, BPW)], mid_v)
        copies = []
        for j in range(NCH):
            copies.append(pltpu.async_copy(
                st_hbm.at[tid_v.at[pl.ds(j * CHUNK, CHUNK)]],
                tval_v.at[pl.ds(j * CHUNK, CHUNK)], sem))
            copies.append(pltpu.async_copy(
                sm_hbm.at[mid_v.at[pl.ds(j * CHUNK, CHUNK)]],
                mval_v.at[pl.ds(j * CHUNK, CHUNK)], sem))
        for c in copies:
            c.wait()
        for g in range(BPW // 16):
            y_v[pl.ds(g * 16, 16)] = (tval_v[pl.ds(g * 16, 16)]
                                      + mval_v[pl.ds(g * 16, 16)])
        pltpu.sync_copy(y_v, out_hbm.at[pl.ds(base, BPW)])

    return k(score_t, score_m, tid, mid)


def _head_body(b_ref, wh_ref, h_ref, o_ref):
    o_ref[...] = jnp.sum(h_ref[...] * wh_ref[...], axis=1) + b_ref[0]


def _head(h, wh, b):
    return pl.pallas_call(
        _head_body,
        grid=(BATCH // BS,),
        in_specs=[
            pl.BlockSpec(memory_space=pltpu.SMEM),
            pl.BlockSpec((1, N_HIDDEN), lambda i: (0, 0)),
            pl.BlockSpec((BS, N_HIDDEN), lambda i: (i, 0)),
        ],
        out_specs=pl.BlockSpec((BS,), lambda i: (i,)),
        out_shape=jax.ShapeDtypeStruct((BATCH,), jnp.float32),
    )(b, wh, h)


def _add_body(a_ref, b_ref, o_ref):
    o_ref[...] = a_ref[...] + b_ref[...]


def _final_add(y_h, y_tm):
    return pl.pallas_call(
        _add_body,
        out_shape=jax.ShapeDtypeStruct((BATCH,), jnp.float32),
    )(y_h, y_tm)


def kernel(h, teacher_id, materia_id, teacher_emb, materia_emb, W, b):
    tid = teacher_id.astype(jnp.int32)
    mid = materia_id.astype(jnp.int32)
    wh = W[:, :N_HIDDEN]
    wtc = W[0, N_HIDDEN:N_HIDDEN + EMB].reshape(EMB, 1)
    wmc = W[0, N_HIDDEN + EMB:].reshape(EMB, 1)
    score_t = _scores(teacher_emb.T, wtc, N_TEACH, BSC)
    score_m = _scores(materia_emb.T, wmc, N_MAT_PAD, N_MAT_PAD)
    y_tm = _sc_gather_scores(score_t, score_m, tid, mid)
    y_h = _head(h, wh, b)
    return _final_add(y_h, y_tm)


# speedup vs baseline: 4.1640x; 1.2813x over previous
import functools

import jax
import jax.numpy as jnp
from jax import lax
from jax.experimental import pallas as pl
from jax.experimental.pallas import tpu as pltpu
from jax.experimental.pallas import tpu_sc as plsc

N_HIDDEN = 128
EMB = 16
BATCH = 16384
N_TEACH = 100000
N_MAT = 1000
N_MAT_PAD = 1024
NW = 32
BPW = BATCH // NW
CHUNK = 128
NCH = BPW // CHUNK
BSC = 25600
BS = 2048


def _score_body(off, w_ref, tT_ref, o_ref):
    wc = w_ref[0, off:off + EMB].reshape(EMB, 1)
    o_ref[...] = jnp.sum(tT_ref[...] * wc, axis=0)


def _scores(tT, w, off, n, blk):
    grid = (n + blk - 1) // blk
    return pl.pallas_call(
        functools.partial(_score_body, off),
        grid=(grid,),
        in_specs=[
            pl.BlockSpec((1, N_HIDDEN + 2 * EMB), lambda i: (0, 0)),
            pl.BlockSpec((EMB, blk), lambda i: (0, i)),
        ],
        out_specs=pl.BlockSpec((blk,), lambda i: (i,)),
        out_shape=jax.ShapeDtypeStruct((n,), jnp.float32),
    )(w, tT)


def _sc_gather_scores(score_t, score_m, tid, mid):
    mesh = plsc.VectorSubcoreMesh(core_axis_name="c", subcore_axis_name="s")

    @functools.partial(
        pl.kernel,
        mesh=mesh,
        compiler_params=pltpu.CompilerParams(
            use_tc_tiling_on_sc=False, needs_layout_passes=False),
        out_type=jax.ShapeDtypeStruct((BATCH,), jnp.float32),
        scratch_types=[
            pltpu.VMEM((BPW,), jnp.int32),
            pltpu.VMEM((BPW,), jnp.int32),
            pltpu.VMEM((BPW,), jnp.float32),
            pltpu.VMEM((N_MAT_PAD,), jnp.float32),
            pltpu.VMEM((BPW,), jnp.float32),
            pltpu.SemaphoreType.DMA,
        ],
    )
    def k(st_hbm, sm_hbm, tid_hbm, mid_hbm, out_hbm,
          tid_v, mid_v, tval_v, sm_v, y_v, sem):
        wid = lax.axis_index("s") * 2 + lax.axis_index("c")
        base = wid * BPW
        pltpu.sync_copy(tid_hbm.at[pl.ds(base, BPW)], tid_v)
        pltpu.sync_copy(mid_hbm.at[pl.ds(base, BPW)], mid_v)
        pltpu.sync_copy(sm_hbm, sm_v)
        copies = []
        for j in range(NCH):
            copies.append(pltpu.async_copy(
                st_hbm.at[tid_v.at[pl.ds(j * CHUNK, CHUNK)]],
                tval_v.at[pl.ds(j * CHUNK, CHUNK)], sem))
        for c in copies:
            c.wait()
        for g in range(BPW // 16):
            mval = plsc.load_gather(sm_v, [mid_v[pl.ds(g * 16, 16)]])
            y_v[pl.ds(g * 16, 16)] = tval_v[pl.ds(g * 16, 16)] + mval
        pltpu.sync_copy(y_v, out_hbm.at[pl.ds(base, BPW)])

    return k(score_t, score_m, tid, mid)


def _head_body(b_ref, w_ref, h_ref, o_ref):
    whc = w_ref[0, :N_HIDDEN].reshape(N_HIDDEN, 1)
    yh = jax.lax.dot_general(h_ref[...], whc, (((1,), (0,)), ((), ())),
                             preferred_element_type=jnp.float32)
    o_ref[...] = yh.reshape(BS) + b_ref[0]


def _head(h, w, b):
    return pl.pallas_call(
        _head_body,
        grid=(BATCH // BS,),
        in_specs=[
            pl.BlockSpec(memory_space=pltpu.SMEM),
            pl.BlockSpec((1, N_HIDDEN + 2 * EMB), lambda i: (0, 0)),
            pl.BlockSpec((BS, N_HIDDEN), lambda i: (i, 0)),
        ],
        out_specs=pl.BlockSpec((BS,), lambda i: (i,)),
        out_shape=jax.ShapeDtypeStruct((BATCH,), jnp.float32),
    )(b, w, h)


def _add_body(a_ref, b_ref, o_ref):
    o_ref[...] = a_ref[...] + b_ref[...]


def _final_add(y_h, y_tm):
    return pl.pallas_call(
        _add_body,
        out_shape=jax.ShapeDtypeStruct((BATCH,), jnp.float32),
    )(y_h, y_tm)


def kernel(h, teacher_id, materia_id, teacher_emb, materia_emb, W, b):
    tid = teacher_id.astype(jnp.int32)
    mid = materia_id.astype(jnp.int32)
    score_t = _scores(teacher_emb.T, W, N_HIDDEN, N_TEACH, BSC)
    score_m = _scores(materia_emb.T, W, N_HIDDEN + EMB, N_MAT_PAD, N_MAT_PAD)
    y_tm = _sc_gather_scores(score_t, score_m, tid, mid)
    y_h = _head(h, W, b)
    return _final_add(y_h, y_tm)


# speedup vs baseline: 4.5529x; 1.0934x over previous
import functools

import jax
import jax.numpy as jnp
from jax import lax
from jax.experimental import pallas as pl
from jax.experimental.pallas import tpu as pltpu
from jax.experimental.pallas import tpu_sc as plsc

N_HIDDEN = 128
EMB = 16
BATCH = 16384
N_TEACH = 100000
N_MAT = 1000
N_MAT_PAD = 1024
NW = 32
BPW = BATCH // NW
CHUNK = 128
NCH = BPW // CHUNK
BSC = 25600
BS = 4096


def _score_body(w_ref, tT_ref, mT_ref, ot_ref, om_ref):
    wt = w_ref[0, N_HIDDEN:N_HIDDEN + EMB].reshape(EMB, 1)
    wm = w_ref[0, N_HIDDEN + EMB:].reshape(EMB, 1)
    ot_ref[...] = jnp.sum(tT_ref[...] * wt, axis=0)
    om_ref[...] = jnp.sum(mT_ref[...] * wm, axis=0)


def _scores(tT, mT):
    del tT, mT
    grid = (N_TEACH + BSC - 1) // BSC
    return pl.pallas_call(
        _score_body,
        grid=(grid,),
        in_specs=[
            pl.BlockSpec((1, N_HIDDEN + 2 * EMB), lambda i: (0, 0)),
            pl.BlockSpec((EMB, BSC), lambda i: (0, i)),
            pl.BlockSpec((EMB, N_MAT_PAD), lambda i: (0, 0)),
        ],
        out_specs=(pl.BlockSpec((BSC,), lambda i: (i,)),
                   pl.BlockSpec((N_MAT_PAD,), lambda i: (0,))),
        out_shape=(jax.ShapeDtypeStruct((N_TEACH,), jnp.float32),
                   jax.ShapeDtypeStruct((N_MAT_PAD,), jnp.float32)),
    )


def _sc_gather_scores(score_t, score_m, tid, mid):
    mesh = plsc.VectorSubcoreMesh(core_axis_name="c", subcore_axis_name="s")

    @functools.partial(
        pl.kernel,
        mesh=mesh,
        compiler_params=pltpu.CompilerParams(
            use_tc_tiling_on_sc=False, needs_layout_passes=False),
        out_type=jax.ShapeDtypeStruct((BATCH,), jnp.float32),
        scratch_types=[
            pltpu.VMEM((BPW,), jnp.int32),
            pltpu.VMEM((BPW,), jnp.int32),
            pltpu.VMEM((BPW,), jnp.float32),
            pltpu.VMEM((N_MAT_PAD,), jnp.float32),
            pltpu.VMEM((BPW,), jnp.float32),
            pltpu.SemaphoreType.DMA,
        ],
    )
    def k(st_hbm, sm_hbm, tid_hbm, mid_hbm, out_hbm,
          tid_v, mid_v, tval_v, sm_v, y_v, sem):
        wid = lax.axis_index("s") * 2 + lax.axis_index("c")
        base = wid * BPW
        pltpu.sync_copy(tid_hbm.at[pl.ds(base, BPW)], tid_v)
        pltpu.sync_copy(mid_hbm.at[pl.ds(base, BPW)], mid_v)
        pltpu.sync_copy(sm_hbm, sm_v)
        copies = []
        for j in range(NCH):
            copies.append(pltpu.async_copy(
                st_hbm.at[tid_v.at[pl.ds(j * CHUNK, CHUNK)]],
                tval_v.at[pl.ds(j * CHUNK, CHUNK)], sem))
        for c in copies:
            c.wait()
        for g in range(BPW // 16):
            mval = plsc.load_gather(sm_v, [mid_v[pl.ds(g * 16, 16)]])
            y_v[pl.ds(g * 16, 16)] = tval_v[pl.ds(g * 16, 16)] + mval
        pltpu.sync_copy(y_v, out_hbm.at[pl.ds(base, BPW)])

    return k(score_t, score_m, tid, mid)


def _head_body(b_ref, w_ref, h_ref, o_ref):
    whc = w_ref[0, :N_HIDDEN].reshape(N_HIDDEN, 1)
    yh = jax.lax.dot_general(h_ref[...], whc, (((1,), (0,)), ((), ())),
                             preferred_element_type=jnp.float32)
    o_ref[...] = yh.reshape(BS) + b_ref[0]


def _head(h, w, b):
    return pl.pallas_call(
        _head_body,
        grid=(BATCH // BS,),
        in_specs=[
            pl.BlockSpec(memory_space=pltpu.SMEM),
            pl.BlockSpec((1, N_HIDDEN + 2 * EMB), lambda i: (0, 0)),
            pl.BlockSpec((BS, N_HIDDEN), lambda i: (i, 0)),
        ],
        out_specs=pl.BlockSpec((BS,), lambda i: (i,)),
        out_shape=jax.ShapeDtypeStruct((BATCH,), jnp.float32),
    )(b, w, h)


def _add_body(a_ref, b_ref, o_ref):
    o_ref[...] = a_ref[...] + b_ref[...]


def _final_add(y_h, y_tm):
    return pl.pallas_call(
        _add_body,
        out_shape=jax.ShapeDtypeStruct((BATCH,), jnp.float32),
    )(y_h, y_tm)


def kernel(h, teacher_id, materia_id, teacher_emb, materia_emb, W, b):
    tid = teacher_id.astype(jnp.int32)
    mid = materia_id.astype(jnp.int32)
    score_t, score_m = _scores(teacher_emb.T, materia_emb.T)(
        W, teacher_emb.T, materia_emb.T)
    y_tm = _sc_gather_scores(score_t, score_m, tid, mid)
    y_h = _head(h, W, b)
    return _final_add(y_h, y_tm)
